# baseline (speedup 1.0000x reference)
import jax
import jax.numpy as jnp
from jax import lax
from jax.experimental import pallas as pl
from jax.experimental.pallas import tpu as pltpu
from jax.experimental.pallas import tpu_sc as plsc

N_NODES = 10000
N_PAD = 10240
D = 128
E = 320000
B = 16384

NC, NS = 2, 16
NW = NC * NS
E_PER_W = E // NW
CH = 80
NCH = E_PER_W // CH
ROWS_PER_TILE = N_PAD // NS

PAIRS_PER_W = B // NW
SCH = 128
NSCH = PAIRS_PER_W // SCH

_mesh = plsc.VectorSubcoreMesh(core_axis_name="c", subcore_axis_name="s")
_sc_params = pltpu.CompilerParams(use_tc_tiling_on_sc=False)
_sc_params_nl = pltpu.CompilerParams(use_tc_tiling_on_sc=False,
                                     needs_layout_passes=False)


def _agg_body(x_hbm, ei_hbm, edst_hbm, feat_hbm, deg_hbm,
              feat_sh, deg_sh, idx_s, idx_d, rows0, rows1, degz, ones_v,
              sem0, sem1, sem_d):
    cid = lax.axis_index("c")
    sid = lax.axis_index("s")
    wid = cid * NS + sid

    zero = jnp.zeros((16,), jnp.float32)

    @pl.loop(0, CH)
    def _(r):
        for k in range(D // 16):
            rows0[r, pl.ds(16 * k, 16)] = zero

    @pl.loop(0, ROWS_PER_TILE // 16)
    def _(r):
        degz[pl.ds(16 * r, 16)] = zero

    for k in range(CH // 16):
        ones_v[pl.ds(16 * k, 16)] = jnp.ones((16,), jnp.float32)

    row0 = sid * ROWS_PER_TILE
    for b in range(ROWS_PER_TILE // CH):
        pltpu.sync_copy(rows0, feat_sh.at[pl.ds(row0 + b * CH, CH)])
    pltpu.sync_copy(degz, deg_sh.at[pl.ds(row0, ROWS_PER_TILE)])
    plsc.subcore_barrier()

    pltpu.sync_copy(ei_hbm.at[0, pl.ds(wid * E_PER_W, E_PER_W)], idx_s)
    pltpu.sync_copy(edst_hbm.at[wid], idx_d)

    def start(c, buf, sem):
        pltpu.async_copy(x_hbm.at[idx_s.at[pl.ds(c * CH, CH)]], buf, sem)

    def wait(buf, sem):
        pltpu.make_async_copy(x_hbm.at[idx_s.at[pl.ds(0, CH)]],
                              buf, sem).wait()

    def scat(c, buf):
        pltpu.sync_copy(buf, feat_sh.at[idx_d.at[c]], add=True)
        pltpu.async_copy(ones_v, deg_sh.at[idx_d.at[c]], sem_d, add=True)

    def wait_deg():
        pltpu.make_async_copy(ones_v, deg_sh.at[idx_d.at[0]], sem_d).wait()

    start(0, rows0, sem0)

    @pl.loop(0, NCH // 2)
    def _(j):
        c0 = 2 * j
        start(c0 + 1, rows1, sem1)
        wait(rows0, sem0)
        scat(c0, rows0)
        start(c0 + 2, rows0, sem0)
        wait(rows1, sem1)
        scat(c0 + 1, rows1)

        @pl.when(j >= 2)
        def _():
            wait_deg()
            wait_deg()

    wait(rows0, sem0)
    scat(NCH - 1, rows0)
    for _ in range(5):
        wait_deg()

    plsc.subcore_barrier()
    pltpu.sync_copy(feat_sh.at[pl.ds(row0, ROWS_PER_TILE)],
                    feat_hbm.at[cid, pl.ds(row0, ROWS_PER_TILE)])
    pltpu.sync_copy(deg_sh.at[pl.ds(row0, ROWS_PER_TILE)],
                    deg_hbm.at[cid, pl.ds(row0, ROWS_PER_TILE)])


_agg_call = pl.kernel(
    _agg_body,
    out_type=[
        jax.ShapeDtypeStruct((NC, N_PAD, D), jnp.float32),
        jax.ShapeDtypeStruct((NC, N_PAD), jnp.float32),
    ],
    mesh=_mesh,
    scratch_types=[
        pltpu.VMEM_SHARED((N_PAD, D), jnp.float32),
        pltpu.VMEM_SHARED((N_PAD,), jnp.float32),
        pltpu.VMEM((E_PER_W,), jnp.int32),
        pltpu.VMEM((NCH, CH), jnp.int32),
        pltpu.VMEM((CH, D), jnp.float32),
        pltpu.VMEM((CH, D), jnp.float32),
        pltpu.VMEM((ROWS_PER_TILE,), jnp.float32),
        pltpu.VMEM((CH,), jnp.float32),
        pltpu.SemaphoreType.DMA,
        pltpu.SemaphoreType.DMA,
        pltpu.SemaphoreType.DMA,
    ],
    compiler_params=_sc_params,
)


RB = 1000


def _densea_body(x_ref, ws_ref, xs_ref):
    xs_ref[...] = jnp.dot(x_ref[...], ws_ref[...],
                          preferred_element_type=jnp.float32)


_densea_call = pl.pallas_call(
    _densea_body,
    grid=(N_NODES // RB,),
    in_specs=[
        pl.BlockSpec((RB, D), lambda i: (i, 0)),
        pl.BlockSpec((D, D), lambda i: (0, 0)),
    ],
    out_specs=pl.BlockSpec((RB, D), lambda i: (i, 0)),
    out_shape=jax.ShapeDtypeStruct((N_NODES, D), jnp.float32),
)


def _denseb_body(feat_ref, deg_ref, xs_ref, wn_ref, h_ref):
    a = feat_ref[0] + feat_ref[1]
    deg = deg_ref[:, 0:1] + deg_ref[:, 1:2]
    mean = a / jnp.maximum(deg, 1.0)
    h_ref[...] = jnp.maximum(
        xs_ref[...]
        + jnp.dot(mean, wn_ref[...], preferred_element_type=jnp.float32),
        0.0)


_denseb_call = pl.pallas_call(
    _denseb_body,
    grid=(N_NODES // RB,),
    in_specs=[
        pl.BlockSpec((NC, RB, D), lambda i: (0, i, 0)),
        pl.BlockSpec((RB, NC), lambda i: (i, 0)),
        pl.BlockSpec((RB, D), lambda i: (i, 0)),
        pl.BlockSpec((D, D), lambda i: (0, 0)),
    ],
    out_specs=pl.BlockSpec((RB, D), lambda i: (i, 0)),
    out_shape=jax.ShapeDtypeStruct((N_NODES, D), jnp.float32),
)


def _pairgather_body(h_hbm, src_hbm, dst_hbm, nb_hbm,
                     us_hbm, vd_hbm, bsum_hbm,
                     idx_s, idx_d, hs0, hd0, hs1, hd1, nb_v, bias_v,
                     sem0, sem1, sem2, sem3):
    cid = lax.axis_index("c")
    sid = lax.axis_index("s")
    wid = cid * NS + sid

    pltpu.sync_copy(src_hbm.at[wid], idx_s)
    pltpu.sync_copy(dst_hbm.at[wid], idx_d)
    pltpu.sync_copy(nb_hbm, nb_v)

    bufs = [(hs0, hd0, sem0, sem1), (hs1, hd1, sem2, sem3)]

    def startg(c, hs, hd, ss, sd):
        pltpu.async_copy(h_hbm.at[idx_s.at[c]], hs, ss)
        pltpu.async_copy(h_hbm.at[idx_d.at[c]], hd, sd)

    def waitg(hs, hd, ss, sd):
        pltpu.make_async_copy(h_hbm.at[idx_s.at[0]], hs, ss).wait()
        pltpu.make_async_copy(h_hbm.at[idx_d.at[0]], hd, sd).wait()

    one = jnp.ones((16,), jnp.int32)
    startg(0, *bufs[0])
    for c in range(NSCH):
        if c + 1 < NSCH:
            startg(c + 1, *bufs[(c + 1) % 2])
        for g in range(SCH // 16):
            sv = idx_s[c, pl.ds(16 * g, 16)] + one
            dv = idx_d[c, pl.ds(16 * g, 16)] + one
            bs = plsc.load_gather(nb_v, [sv])
            bd = plsc.load_gather(nb_v, [dv])
            bias_v[pl.ds(16 * g, 16)] = bs + bd
        hs, hd, ss, sd = bufs[c % 2]
        waitg(hs, hd, ss, sd)
        base = wid * PAIRS_PER_W + c * SCH
        pltpu.sync_copy(hs, us_hbm.at[pl.ds(base, SCH)])
        pltpu.sync_copy(hd, vd_hbm.at[pl.ds(base, SCH)])
        pltpu.sync_copy(bias_v, bsum_hbm.at[pl.ds(base, SCH)])


_pairgather_call = pl.kernel(
    _pairgather_body,
    out_type=[
        jax.ShapeDtypeStruct((B, D), jnp.float32),
        jax.ShapeDtypeStruct((B, D), jnp.float32),
        jax.ShapeDtypeStruct((B,), jnp.float32),
    ],
    mesh=_mesh,
    scratch_types=[
        pltpu.VMEM((NSCH, SCH), jnp.int32),
        pltpu.VMEM((NSCH, SCH), jnp.int32),
        pltpu.VMEM((SCH, D), jnp.float32),
        pltpu.VMEM((SCH, D), jnp.float32),
        pltpu.VMEM((SCH, D), jnp.float32),
        pltpu.VMEM((SCH, D), jnp.float32),
        pltpu.VMEM((N_NODES + 1,), jnp.float32),
        pltpu.VMEM((SCH,), jnp.float32),
        pltpu.SemaphoreType.DMA,
        pltpu.SemaphoreType.DMA,
        pltpu.SemaphoreType.DMA,
        pltpu.SemaphoreType.DMA,
    ],
    compiler_params=_sc_params_nl,
)


SB = 2048


def _score_body(us_ref, vd_ref, bsum_ref, out_ref):
    out_ref[...] = jnp.sum(us_ref[...] * vd_ref[...], axis=1) + bsum_ref[...]


_score_call = pl.pallas_call(
    _score_body,
    grid=(B // SB,),
    in_specs=[
        pl.BlockSpec((SB, D), lambda i: (i, 0)),
        pl.BlockSpec((SB, D), lambda i: (i, 0)),
        pl.BlockSpec((SB,), lambda i: (i,)),
    ],
    out_specs=pl.BlockSpec((SB,), lambda i: (i,)),
    out_shape=jax.ShapeDtypeStruct((B,), jnp.float32),
)


def kernel(x, edge_index, src, dst, W_self, W_neigh, node_biases):
    x = x.astype(jnp.float32)
    ei = edge_index.astype(jnp.int32)
    e_dst = ei[1].reshape(NW, NCH, CH)
    src3 = src.astype(jnp.int32).reshape(NW, NSCH, SCH)
    dst3 = dst.astype(jnp.int32).reshape(NW, NSCH, SCH)
    nb = node_biases.astype(jnp.float32)

    feat, deg = _agg_call(x, ei, e_dst)
    xs = _densea_call(x, W_self)
    h = _denseb_call(feat, deg.T, xs, W_neigh)
    us, vd, bsum = _pairgather_call(h, src3, dst3, nb)
    return _score_call(us, vd, bsum)


# speedup vs baseline: 1.0791x; 1.0791x over previous
import jax
import jax.numpy as jnp
from jax import lax
from jax.experimental import pallas as pl
from jax.experimental.pallas import tpu as pltpu
from jax.experimental.pallas import tpu_sc as plsc

N_NODES = 10000
N_PAD = 10240
D = 128
E = 320000
B = 16384

NC, NS = 2, 16
NW = NC * NS
E_PER_W = E // NW
CH = 80
NCH = E_PER_W // CH
ROWS_PER_TILE = N_PAD // NS

PAIRS_PER_W = B // NW
SCH = 128
NSCH = PAIRS_PER_W // SCH

_mesh = plsc.VectorSubcoreMesh(core_axis_name="c", subcore_axis_name="s")
_sc_params = pltpu.CompilerParams(use_tc_tiling_on_sc=False)
_sc_params_nl = pltpu.CompilerParams(use_tc_tiling_on_sc=False,
                                     needs_layout_passes=False)


def _agg_body(x_hbm, ei_hbm, feat_hbm, deg_hbm,
              feat_sh, deg_sh, idx_s, idx_d, rows0, rows1, degz, ones_v,
              sem0, sem1, sem_d):
    cid = lax.axis_index("c")
    sid = lax.axis_index("s")
    wid = cid * NS + sid

    zero = jnp.zeros((16,), jnp.float32)

    @pl.loop(0, CH)
    def _(r):
        for k in range(D // 16):
            rows0[r, pl.ds(16 * k, 16)] = zero

    @pl.loop(0, ROWS_PER_TILE // 16)
    def _(r):
        degz[pl.ds(16 * r, 16)] = zero

    for k in range(CH // 16):
        ones_v[pl.ds(16 * k, 16)] = jnp.ones((16,), jnp.float32)

    row0 = sid * ROWS_PER_TILE
    for b in range(ROWS_PER_TILE // CH):
        pltpu.sync_copy(rows0, feat_sh.at[pl.ds(row0 + b * CH, CH)])
    pltpu.sync_copy(degz, deg_sh.at[pl.ds(row0, ROWS_PER_TILE)])
    plsc.subcore_barrier()

    pltpu.sync_copy(ei_hbm.at[0, wid], idx_s)
    pltpu.sync_copy(ei_hbm.at[1, wid], idx_d)

    def start(c, buf, sem):
        pltpu.async_copy(x_hbm.at[idx_s.at[c]], buf, sem)

    def wait(buf, sem):
        pltpu.make_async_copy(x_hbm.at[idx_s.at[0]], buf, sem).wait()

    def scat(c, buf):
        pltpu.sync_copy(buf, feat_sh.at[idx_d.at[c]], add=True)
        pltpu.async_copy(ones_v, deg_sh.at[idx_d.at[c]], sem_d, add=True)

    def wait_deg():
        pltpu.make_async_copy(ones_v, deg_sh.at[idx_d.at[0]], sem_d).wait()

    start(0, rows0, sem0)

    @pl.loop(0, NCH // 2)
    def _(j):
        c0 = 2 * j
        start(c0 + 1, rows1, sem1)
        wait(rows0, sem0)
        scat(c0, rows0)
        start(c0 + 2, rows0, sem0)
        wait(rows1, sem1)
        scat(c0 + 1, rows1)

        @pl.when(j >= 2)
        def _():
            wait_deg()
            wait_deg()

    wait(rows0, sem0)
    scat(NCH - 1, rows0)
    for _ in range(5):
        wait_deg()

    plsc.subcore_barrier()
    pltpu.sync_copy(feat_sh.at[pl.ds(row0, ROWS_PER_TILE)],
                    feat_hbm.at[cid, pl.ds(row0, ROWS_PER_TILE)])
    pltpu.sync_copy(deg_sh.at[pl.ds(row0, ROWS_PER_TILE)],
                    deg_hbm.at[cid, pl.ds(row0, ROWS_PER_TILE)])


_agg_call = pl.kernel(
    _agg_body,
    out_type=[
        jax.ShapeDtypeStruct((NC, N_PAD, D), jnp.float32),
        jax.ShapeDtypeStruct((NC, N_PAD), jnp.float32),
    ],
    mesh=_mesh,
    scratch_types=[
        pltpu.VMEM_SHARED((N_PAD, D), jnp.float32),
        pltpu.VMEM_SHARED((N_PAD,), jnp.float32),
        pltpu.VMEM((NCH, CH), jnp.int32),
        pltpu.VMEM((NCH, CH), jnp.int32),
        pltpu.VMEM((CH, D), jnp.float32),
        pltpu.VMEM((CH, D), jnp.float32),
        pltpu.VMEM((ROWS_PER_TILE,), jnp.float32),
        pltpu.VMEM((CH,), jnp.float32),
        pltpu.SemaphoreType.DMA,
        pltpu.SemaphoreType.DMA,
        pltpu.SemaphoreType.DMA,
    ],
    compiler_params=_sc_params,
)


RB = 1000


def _densea_body(x_ref, ws_ref, xs_ref):
    xs_ref[...] = jnp.dot(x_ref[...], ws_ref[...],
                          preferred_element_type=jnp.float32)


_densea_call = pl.pallas_call(
    _densea_body,
    grid=(N_NODES // RB,),
    in_specs=[
        pl.BlockSpec((RB, D), lambda i: (i, 0)),
        pl.BlockSpec((D, D), lambda i: (0, 0)),
    ],
    out_specs=pl.BlockSpec((RB, D), lambda i: (i, 0)),
    out_shape=jax.ShapeDtypeStruct((N_NODES, D), jnp.float32),
)


def _denseb_body(feat_ref, deg_ref, xs_ref, wn_ref, h_ref):
    a = feat_ref[0] + feat_ref[1]
    deg = deg_ref[:, 0:1] + deg_ref[:, 1:2]
    mean = a / jnp.maximum(deg, 1.0)
    h_ref[...] = jnp.maximum(
        xs_ref[...]
        + jnp.dot(mean, wn_ref[...], preferred_element_type=jnp.float32),
        0.0)


_denseb_call = pl.pallas_call(
    _denseb_body,
    grid=(N_NODES // RB,),
    in_specs=[
        pl.BlockSpec((NC, RB, D), lambda i: (0, i, 0)),
        pl.BlockSpec((RB, NC), lambda i: (i, 0)),
        pl.BlockSpec((RB, D), lambda i: (i, 0)),
        pl.BlockSpec((D, D), lambda i: (0, 0)),
    ],
    out_specs=pl.BlockSpec((RB, D), lambda i: (i, 0)),
    out_shape=jax.ShapeDtypeStruct((N_NODES, D), jnp.float32),
)


def _pairgather_body(h_hbm, src_hbm, dst_hbm, nb_hbm,
                     us_hbm, vd_hbm, bsum_hbm,
                     idx_s, idx_d, hs0, hd0, hs1, hd1, nb_v, bias_v,
                     sem0, sem1, sem2, sem3):
    cid = lax.axis_index("c")
    sid = lax.axis_index("s")
    wid = cid * NS + sid

    pltpu.sync_copy(src_hbm.at[wid], idx_s)
    pltpu.sync_copy(dst_hbm.at[wid], idx_d)
    pltpu.sync_copy(nb_hbm, nb_v)

    bufs = [(hs0, hd0, sem0, sem1), (hs1, hd1, sem2, sem3)]

    def startg(c, hs, hd, ss, sd):
        pltpu.async_copy(h_hbm.at[idx_s.at[c]], hs, ss)
        pltpu.async_copy(h_hbm.at[idx_d.at[c]], hd, sd)

    def waitg(hs, hd, ss, sd):
        pltpu.make_async_copy(h_hbm.at[idx_s.at[0]], hs, ss).wait()
        pltpu.make_async_copy(h_hbm.at[idx_d.at[0]], hd, sd).wait()

    one = jnp.ones((16,), jnp.int32)
    startg(0, *bufs[0])
    for c in range(NSCH):
        if c + 1 < NSCH:
            startg(c + 1, *bufs[(c + 1) % 2])
        for g in range(SCH // 16):
            sv = idx_s[c, pl.ds(16 * g, 16)] + one
            dv = idx_d[c, pl.ds(16 * g, 16)] + one
            bs = plsc.load_gather(nb_v, [sv])
            bd = plsc.load_gather(nb_v, [dv])
            bias_v[pl.ds(16 * g, 16)] = bs + bd
        hs, hd, ss, sd = bufs[c % 2]
        waitg(hs, hd, ss, sd)
        base = wid * PAIRS_PER_W + c * SCH
        pltpu.sync_copy(hs, us_hbm.at[pl.ds(base, SCH)])
        pltpu.sync_copy(hd, vd_hbm.at[pl.ds(base, SCH)])
        pltpu.sync_copy(bias_v, bsum_hbm.at[pl.ds(base, SCH)])


_pairgather_call = pl.kernel(
    _pairgather_body,
    out_type=[
        jax.ShapeDtypeStruct((B, D), jnp.float32),
        jax.ShapeDtypeStruct((B, D), jnp.float32),
        jax.ShapeDtypeStruct((B,), jnp.float32),
    ],
    mesh=_mesh,
    scratch_types=[
        pltpu.VMEM((NSCH, SCH), jnp.int32),
        pltpu.VMEM((NSCH, SCH), jnp.int32),
        pltpu.VMEM((SCH, D), jnp.float32),
        pltpu.VMEM((SCH, D), jnp.float32),
        pltpu.VMEM((SCH, D), jnp.float32),
        pltpu.VMEM((SCH, D), jnp.float32),
        pltpu.VMEM((N_NODES + 1,), jnp.float32),
        pltpu.VMEM((SCH,), jnp.float32),
        pltpu.SemaphoreType.DMA,
        pltpu.SemaphoreType.DMA,
        pltpu.SemaphoreType.DMA,
        pltpu.SemaphoreType.DMA,
    ],
    compiler_params=_sc_params_nl,
)


SB = 2048


def _score_body(us_ref, vd_ref, bsum_ref, out_ref):
    out_ref[...] = jnp.sum(us_ref[...] * vd_ref[...], axis=1) + bsum_ref[...]


_score_call = pl.pallas_call(
    _score_body,
    grid=(B // SB,),
    in_specs=[
        pl.BlockSpec((SB, D), lambda i: (i, 0)),
        pl.BlockSpec((SB, D), lambda i: (i, 0)),
        pl.BlockSpec((SB,), lambda i: (i,)),
    ],
    out_specs=pl.BlockSpec((SB,), lambda i: (i,)),
    out_shape=jax.ShapeDtypeStruct((B,), jnp.float32),
)


def kernel(x, edge_index, src, dst, W_self, W_neigh, node_biases):
    x = x.astype(jnp.float32)
    ei4 = edge_index.astype(jnp.int32).reshape(2, NW, NCH, CH)
    src3 = src.astype(jnp.int32).reshape(NW, NSCH, SCH)
    dst3 = dst.astype(jnp.int32).reshape(NW, NSCH, SCH)
    nb = node_biases.astype(jnp.float32)

    feat, deg = _agg_call(x, ei4)
    xs = _densea_call(x, W_self)
    h = _denseb_call(feat, deg.T, xs, W_neigh)
    us, vd, bsum = _pairgather_call(h, src3, dst3, nb)
    return _score_call(us, vd, bsum)


# speedup vs baseline: 1.1011x; 1.0203x over previous
import jax
import jax.numpy as jnp
from jax import lax
from jax.experimental import pallas as pl
from jax.experimental.pallas import tpu as pltpu
from jax.experimental.pallas import tpu_sc as plsc

N_NODES = 10000
N_PAD = 10240
D = 128
E = 320000
B = 16384

NC, NS = 2, 16
NW = NC * NS
E_PER_W = E // NW
CH = 80
NCH = E_PER_W // CH
ROWS_PER_TILE = N_PAD // NS

PAIRS_PER_W = B // NW
SCH = 128
NSCH = PAIRS_PER_W // SCH

_mesh = plsc.VectorSubcoreMesh(core_axis_name="c", subcore_axis_name="s")
_sc_params = pltpu.CompilerParams(use_tc_tiling_on_sc=False)
_sc_params_nl = pltpu.CompilerParams(use_tc_tiling_on_sc=False,
                                     needs_layout_passes=False)


def _agg_body(x_hbm, ei_hbm, feat_hbm, deg_hbm,
              feat_sh, deg_sh, idx_s, idx_d, rows0, rows1, degz, ones_v,
              sem0, sem1, sem_d):
    cid = lax.axis_index("c")
    sid = lax.axis_index("s")
    wid = cid * NS + sid

    zero = jnp.zeros((16,), jnp.float32)

    @pl.loop(0, CH)
    def _(r):
        for k in range(D // 16):
            rows0[r, pl.ds(16 * k, 16)] = zero

    @pl.loop(0, ROWS_PER_TILE // 16)
    def _(r):
        degz[pl.ds(16 * r, 16)] = zero

    for k in range(CH // 16):
        ones_v[pl.ds(16 * k, 16)] = jnp.ones((16,), jnp.float32)

    row0 = sid * ROWS_PER_TILE
    for b in range(ROWS_PER_TILE // CH):
        pltpu.sync_copy(rows0, feat_sh.at[pl.ds(row0 + b * CH, CH)])
    pltpu.sync_copy(degz, deg_sh.at[pl.ds(row0, ROWS_PER_TILE)])
    plsc.subcore_barrier()

    pltpu.sync_copy(ei_hbm.at[0, wid], idx_s)
    pltpu.sync_copy(ei_hbm.at[1, wid], idx_d)

    def start(c, buf, sem):
        pltpu.async_copy(x_hbm.at[idx_s.at[c]], buf, sem)

    def wait(buf, sem):
        pltpu.make_async_copy(x_hbm.at[idx_s.at[0]], buf, sem).wait()

    def scat(c, buf):
        pltpu.sync_copy(buf, feat_sh.at[idx_d.at[c]], add=True)
        pltpu.async_copy(ones_v, deg_sh.at[idx_d.at[c]], sem_d, add=True)

    def wait_deg():
        pltpu.make_async_copy(ones_v, deg_sh.at[idx_d.at[0]], sem_d).wait()

    start(0, rows0, sem0)

    @pl.loop(0, NCH // 2)
    def _(j):
        c0 = 2 * j
        start(c0 + 1, rows1, sem1)
        wait(rows0, sem0)
        scat(c0, rows0)
        start(c0 + 2, rows0, sem0)
        wait(rows1, sem1)
        scat(c0 + 1, rows1)

        @pl.when(j >= 2)
        def _():
            wait_deg()
            wait_deg()

    wait(rows0, sem0)
    scat(NCH - 1, rows0)
    for _ in range(5):
        wait_deg()

    plsc.subcore_barrier()
    pltpu.sync_copy(feat_sh.at[pl.ds(row0, ROWS_PER_TILE)],
                    feat_hbm.at[cid, pl.ds(row0, ROWS_PER_TILE)])
    pltpu.sync_copy(deg_sh.at[pl.ds(row0, ROWS_PER_TILE)],
                    deg_hbm.at[cid, pl.ds(row0, ROWS_PER_TILE)])


_agg_call = pl.kernel(
    _agg_body,
    out_type=[
        jax.ShapeDtypeStruct((NC, N_PAD, D), jnp.float32),
        jax.ShapeDtypeStruct((NC, N_PAD), jnp.float32),
    ],
    mesh=_mesh,
    scratch_types=[
        pltpu.VMEM_SHARED((N_PAD, D), jnp.float32),
        pltpu.VMEM_SHARED((N_PAD,), jnp.float32),
        pltpu.VMEM((NCH, CH), jnp.int32),
        pltpu.VMEM((NCH, CH), jnp.int32),
        pltpu.VMEM((CH, D), jnp.float32),
        pltpu.VMEM((CH, D), jnp.float32),
        pltpu.VMEM((ROWS_PER_TILE,), jnp.float32),
        pltpu.VMEM((CH,), jnp.float32),
        pltpu.SemaphoreType.DMA,
        pltpu.SemaphoreType.DMA,
        pltpu.SemaphoreType.DMA,
    ],
    compiler_params=_sc_params,
)


RB = 2000


def _densea_body(x_ref, ws_ref, xs_ref):
    xs_ref[...] = jnp.dot(x_ref[...], ws_ref[...],
                          preferred_element_type=jnp.float32)


_densea_call = pl.pallas_call(
    _densea_body,
    grid=(N_NODES // RB,),
    in_specs=[
        pl.BlockSpec((RB, D), lambda i: (i, 0)),
        pl.BlockSpec((D, D), lambda i: (0, 0)),
    ],
    out_specs=pl.BlockSpec((RB, D), lambda i: (i, 0)),
    out_shape=jax.ShapeDtypeStruct((N_NODES, D), jnp.float32),
)


def _denseb_body(feat_ref, deg_ref, xs_ref, wn_ref, h_ref):
    a = feat_ref[0] + feat_ref[1]
    deg = deg_ref[:, 0:1] + deg_ref[:, 1:2]
    mean = a / jnp.maximum(deg, 1.0)
    h_ref[...] = jnp.maximum(
        xs_ref[...]
        + jnp.dot(mean, wn_ref[...], preferred_element_type=jnp.float32),
        0.0)


_denseb_call = pl.pallas_call(
    _denseb_body,
    grid=(N_NODES // RB,),
    in_specs=[
        pl.BlockSpec((NC, RB, D), lambda i: (0, i, 0)),
        pl.BlockSpec((RB, NC), lambda i: (i, 0)),
        pl.BlockSpec((RB, D), lambda i: (i, 0)),
        pl.BlockSpec((D, D), lambda i: (0, 0)),
    ],
    out_specs=pl.BlockSpec((RB, D), lambda i: (i, 0)),
    out_shape=jax.ShapeDtypeStruct((N_NODES, D), jnp.float32),
)


def _pairgather_body(h_hbm, src_hbm, dst_hbm, nb_hbm,
                     us_hbm, vd_hbm, bsum_hbm,
                     idx_s, idx_d, hs0, hd0, hs1, hd1, hs2, hd2, nb_v, bias_v,
                     sem0, sem1, sem2, sem3, sem4, sem5):
    cid = lax.axis_index("c")
    sid = lax.axis_index("s")
    wid = cid * NS + sid

    pltpu.sync_copy(src_hbm.at[wid], idx_s)
    pltpu.sync_copy(dst_hbm.at[wid], idx_d)
    pltpu.sync_copy(nb_hbm, nb_v)

    bufs = [(hs0, hd0, sem0, sem1), (hs1, hd1, sem2, sem3),
            (hs2, hd2, sem4, sem5)]

    def startg(c, hs, hd, ss, sd):
        pltpu.async_copy(h_hbm.at[idx_s.at[c]], hs, ss)
        pltpu.async_copy(h_hbm.at[idx_d.at[c]], hd, sd)

    def waitg(hs, hd, ss, sd):
        pltpu.make_async_copy(h_hbm.at[idx_s.at[0]], hs, ss).wait()
        pltpu.make_async_copy(h_hbm.at[idx_d.at[0]], hd, sd).wait()

    one = jnp.ones((16,), jnp.int32)
    startg(0, *bufs[0])
    startg(1, *bufs[1])
    startg(2, *bufs[2])
    for c in range(NSCH):
        for g in range(SCH // 16):
            sv = idx_s[c, pl.ds(16 * g, 16)] + one
            dv = idx_d[c, pl.ds(16 * g, 16)] + one
            bs = plsc.load_gather(nb_v, [sv])
            bd = plsc.load_gather(nb_v, [dv])
            bias_v[pl.ds(16 * g, 16)] = bs + bd
        hs, hd, ss, sd = bufs[c % 3]
        waitg(hs, hd, ss, sd)
        base = wid * PAIRS_PER_W + c * SCH
        pltpu.sync_copy(hs, us_hbm.at[pl.ds(base, SCH)])
        pltpu.sync_copy(hd, vd_hbm.at[pl.ds(base, SCH)])
        pltpu.sync_copy(bias_v, bsum_hbm.at[pl.ds(base, SCH)])
        if c + 3 < NSCH:
            startg(c + 3, *bufs[c % 3])


_pairgather_call = pl.kernel(
    _pairgather_body,
    out_type=[
        jax.ShapeDtypeStruct((B, D), jnp.float32),
        jax.ShapeDtypeStruct((B, D), jnp.float32),
        jax.ShapeDtypeStruct((B,), jnp.float32),
    ],
    mesh=_mesh,
    scratch_types=[
        pltpu.VMEM((NSCH, SCH), jnp.int32),
        pltpu.VMEM((NSCH, SCH), jnp.int32),
        pltpu.VMEM((SCH, D), jnp.float32),
        pltpu.VMEM((SCH, D), jnp.float32),
        pltpu.VMEM((SCH, D), jnp.float32),
        pltpu.VMEM((SCH, D), jnp.float32),
        pltpu.VMEM((SCH, D), jnp.float32),
        pltpu.VMEM((SCH, D), jnp.float32),
        pltpu.VMEM((N_NODES + 1,), jnp.float32),
        pltpu.VMEM((SCH,), jnp.float32),
        pltpu.SemaphoreType.DMA,
        pltpu.SemaphoreType.DMA,
        pltpu.SemaphoreType.DMA,
        pltpu.SemaphoreType.DMA,
        pltpu.SemaphoreType.DMA,
        pltpu.SemaphoreType.DMA,
    ],
    compiler_params=_sc_params_nl,
)


SB = 2048


def _score_body(us_ref, vd_ref, bsum_ref, out_ref):
    out_ref[...] = jnp.sum(us_ref[...] * vd_ref[...], axis=1) + bsum_ref[...]


_score_call = pl.pallas_call(
    _score_body,
    grid=(B // SB,),
    in_specs=[
        pl.BlockSpec((SB, D), lambda i: (i, 0)),
        pl.BlockSpec((SB, D), lambda i: (i, 0)),
        pl.BlockSpec((SB,), lambda i: (i,)),
    ],
    out_specs=pl.BlockSpec((SB,), lambda i: (i,)),
    out_shape=jax.ShapeDtypeStruct((B,), jnp.float32),
)


def kernel(x, edge_index, src, dst, W_self, W_neigh, node_biases):
    x = x.astype(jnp.float32)
    ei4 = edge_index.astype(jnp.int32).reshape(2, NW, NCH, CH)
    src3 = src.astype(jnp.int32).reshape(NW, NSCH, SCH)
    dst3 = dst.astype(jnp.int32).reshape(NW, NSCH, SCH)
    nb = node_biases.astype(jnp.float32)

    feat, deg = _agg_call(x, ei4)
    xs = _densea_call(x, W_self)
    h = _denseb_call(feat, deg.T, xs, W_neigh)
    us, vd, bsum = _pairgather_call(h, src3, dst3, nb)
    return _score_call(us, vd, bsum)
